# baseline (speedup 1.0000x reference)
import functools
import math

import numpy as np
import jax
import jax.numpy as jnp
from jax import lax
from jax.experimental import pallas as pl
from jax.experimental.pallas import tpu as pltpu
from jax.experimental.pallas import tpu_sc as plsc

VOCAB = 1000000
D = 64
B = 4096
L = 200
PADW = 128

NC = 2
NS = 16
NW = NC * NS
BW = B // NW
NBUF = 4
LOOKAHEAD = 2

NCH = VOCAB // PADW
CH_REM = NCH % NW
CPW = NCH // NW + 1
VTAIL = NCH * PADW

SCALE = 2.0 * math.sqrt(D)


def _make_pe() -> np.ndarray:
    position = np.arange(0, L, dtype=np.float32)[:, None]
    div_even = np.power(10000.0, np.arange(0, D, 2, dtype=np.float32) / D)
    div_odd = np.power(10000.0, np.arange(1, D, 2, dtype=np.float32) / D)
    pe = np.zeros((L, D), dtype=np.float32)
    pe[:, 0::2] = np.sin(position * div_even)
    pe[:, 1::2] = np.cos(position * div_odd)
    return pe


_PE = _make_pe()


def _mesh():
    return plsc.VectorSubcoreMesh(
        core_axis_name="c", subcore_axis_name="s", num_cores=NC, num_subcores=NS
    )


@functools.cache
def _build_repack():
    @functools.partial(
        pl.kernel,
        mesh=_mesh(),
        out_type=jax.ShapeDtypeStruct((VOCAB, PADW), jnp.float32),
        scratch_types=[
            [pltpu.VMEM((D, PADW), jnp.float32) for _ in range(2)],
            [pltpu.VMEM((PADW, PADW), jnp.float32) for _ in range(2)],
            [pltpu.SemaphoreType.DMA for _ in range(2)],
            [pltpu.SemaphoreType.DMA for _ in range(2)],
        ],
        compiler_params=pltpu.CompilerParams(needs_layout_passes=False),
    )
    def repack(tt_hbm, tail_hbm, out_hbm, panel_v, stg_v, gsem, ssem):
        wid = lax.axis_index("s") * NC + lax.axis_index("c")
        nw = jnp.where(wid < CH_REM, CPW, CPW - 1)

        def chunk_col(k):
            return (wid + NW * k) * PADW

        def start_panel(g, k):
            pltpu.async_copy(
                tt_hbm.at[:, pl.ds(chunk_col(k), PADW)], panel_v[g], gsem[g]
            )

        start_panel(0, 0)
        iota = lax.iota(jnp.int32, 16)
        idx_r = [iota + 16 * g for g in range(PADW // 16)]

        def body(i, carry):
            for j in range(2):
                k = i * 2 + j

                @pl.when(k < nw)
                def _():
                    @pl.when(k + 1 < nw)
                    def _():
                        start_panel((j + 1) % 2, k + 1)

                    pltpu.make_async_copy(
                        tt_hbm.at[:, pl.ds(chunk_col(k), PADW)],
                        panel_v[j],
                        gsem[j],
                    ).wait()

                    @pl.when(k >= 2)
                    def _():
                        pltpu.make_async_copy(
                            stg_v[j],
                            out_hbm.at[pl.ds(chunk_col(k), PADW), :],
                            ssem[j],
                        ).wait()

                    def per_d(d, c):
                        idx_c = jnp.full((16,), d, jnp.int32)
                        for g in range(PADW // 16):
                            v = panel_v[j][d, pl.ds(16 * g, 16)]
                            plsc.store_scatter(stg_v[j], [idx_r[g], idx_c], v)
                        return c

                    lax.fori_loop(0, D, per_d, 0)
                    pltpu.async_copy(
                        stg_v[j], out_hbm.at[pl.ds(chunk_col(k), PADW), :], ssem[j]
                    )
            return carry

        lax.fori_loop(0, (CPW + 1) // 2, body, 0)

        for j in range(2):
            @pl.when(nw >= 2 - j)
            def _():
                pltpu.make_async_copy(
                    stg_v[j], out_hbm.at[pl.ds(0, PADW), :], ssem[j]
                ).wait()

        @pl.when(wid == NW - 1)
        def _():
            pltpu.sync_copy(tail_hbm, stg_v[0].at[pl.ds(0, D), :])
            pltpu.sync_copy(
                stg_v[0].at[pl.ds(0, D), :], out_hbm.at[pl.ds(VTAIL, D), :]
            )

    return repack


@functools.cache
def _build_lookup():
    @functools.partial(
        pl.kernel,
        mesh=_mesh(),
        out_type=jax.ShapeDtypeStruct((L, D, B), jnp.float32),
        scratch_types=[
            [pltpu.VMEM((BW,), jnp.int32) for _ in range(NBUF)],
            [pltpu.VMEM((BW, PADW), jnp.float32) for _ in range(NBUF)],
            [pltpu.VMEM((D, BW), jnp.float32) for _ in range(2)],
            pltpu.VMEM((L, D), jnp.float32),
            [pltpu.SemaphoreType.DMA for _ in range(NBUF)],
            [pltpu.SemaphoreType.DMA for _ in range(2)],
        ],
        compiler_params=pltpu.CompilerParams(needs_layout_passes=False),
    )
    def emb(xt_hbm, table_hbm, pe_hbm, out_hbm, idx_v, rows_v, stg_v, pe_v, gsem, ssem):
        wid = lax.axis_index("s") * NC + lax.axis_index("c")
        b0 = wid * BW
        pltpu.sync_copy(pe_hbm, pe_v)

        def start_gather(g, l):
            pltpu.sync_copy(xt_hbm.at[l, pl.ds(b0, BW)], idx_v[g])
            pltpu.async_copy(table_hbm.at[idx_v[g]], rows_v[g], gsem[g])

        for j in range(LOOKAHEAD):
            start_gather(j, j)

        iota = lax.iota(jnp.int32, 16)
        idx_d = [iota + 16 * k for k in range(D // 16)]

        def body(i, carry):
            for j in range(NBUF):
                l = i * NBUF + j
                g = j
                ga = (j + LOOKAHEAD) % NBUF
                st = j % 2

                @pl.when(l + LOOKAHEAD < L)
                def _():
                    start_gather(ga, l + LOOKAHEAD)

                pltpu.make_async_copy(
                    table_hbm.at[idx_v[g]], rows_v[g], gsem[g]
                ).wait()

                @pl.when(l >= 2)
                def _():
                    pltpu.make_async_copy(
                        stg_v[st], out_hbm.at[l, :, pl.ds(b0, BW)], ssem[st]
                    ).wait()

                pe_row = [pe_v[l, pl.ds(16 * k, 16)] for k in range(D // 16)]

                def per_b(b4, c):
                    for bi in range(4):
                        b = b4 * 4 + bi
                        idx_b = jnp.full((16,), b, jnp.int32)
                        for k in range(D // 16):
                            v = rows_v[g][b, pl.ds(16 * k, 16)] * SCALE + pe_row[k]
                            plsc.store_scatter(stg_v[st], [idx_d[k], idx_b], v)
                    return c

                lax.fori_loop(0, BW // 4, per_b, 0)
                pltpu.async_copy(
                    stg_v[st], out_hbm.at[l, :, pl.ds(b0, BW)], ssem[st]
                )
            return carry

        lax.fori_loop(0, L // NBUF, body, 0)

        for st in range(2):
            pltpu.make_async_copy(
                stg_v[st], out_hbm.at[L - 2 + st, :, pl.ds(b0, BW)], ssem[st]
            ).wait()

    return emb


def kernel(x, table):
    xt = jnp.transpose(x)
    tt = jnp.transpose(table)
    tail = jnp.pad(table[VTAIL:, :], ((0, 0), (0, PADW - D)))
    table_p = _build_repack()(tt, tail)
    outp = _build_lookup()(xt, table_p, _PE)
    return jnp.transpose(outp, (2, 0, 1))


# speedup vs baseline: 1.1747x; 1.1747x over previous
import functools
import math

import numpy as np
import jax
import jax.numpy as jnp
from jax import lax
from jax.experimental import pallas as pl
from jax.experimental.pallas import tpu as pltpu
from jax.experimental.pallas import tpu_sc as plsc

VOCAB = 1000000
D = 64
B = 4096
L = 200
PADW = 128
VC = 512

NC = 2
NS = 16
NW = NC * NS
SEQ_PER_W = B // NW
CH = 100
NBUF = 4
LOOKAHEAD = 2

SCALE = 2.0 * math.sqrt(D)


def _make_pe() -> np.ndarray:
    position = np.arange(0, L, dtype=np.float32)[:, None]
    div_even = np.power(10000.0, np.arange(0, D, 2, dtype=np.float32) / D)
    div_odd = np.power(10000.0, np.arange(1, D, 2, dtype=np.float32) / D)
    pe = np.zeros((L, D), dtype=np.float32)
    pe[:, 0::2] = np.sin(position * div_even)
    pe[:, 1::2] = np.cos(position * div_odd)
    return pe


_PE = _make_pe()


def _repack_block(tt_ref, out_ref):
    t = jnp.transpose(tt_ref[...])
    out_ref[...] = jnp.concatenate(
        [t, jnp.zeros((t.shape[0], PADW - D), jnp.float32)], axis=1
    )


@functools.cache
def _build_repack():
    grid = (VOCAB + VC - 1) // VC
    return pl.pallas_call(
        _repack_block,
        grid=(grid,),
        in_specs=[pl.BlockSpec((D, VC), lambda i: (0, i))],
        out_specs=pl.BlockSpec((VC, PADW), lambda i: (i, 0)),
        out_shape=jax.ShapeDtypeStruct((VOCAB, PADW), jnp.float32),
    )


@functools.cache
def _build_lookup():
    mesh = plsc.VectorSubcoreMesh(
        core_axis_name="c", subcore_axis_name="s", num_cores=NC, num_subcores=NS
    )

    @functools.partial(
        pl.kernel,
        mesh=mesh,
        out_type=jax.ShapeDtypeStruct((B, L, D), jnp.float32),
        scratch_types=[
            [pltpu.VMEM((L,), jnp.int32) for _ in range(2)],
            [pltpu.VMEM((L, PADW), jnp.float32) for _ in range(2)],
            [pltpu.VMEM((L, D), jnp.float32) for _ in range(2)],
            pltpu.VMEM((L * D,), jnp.float32),
            [pltpu.SemaphoreType.DMA for _ in range(2)],
            [pltpu.SemaphoreType.DMA for _ in range(2)],
        ],
        compiler_params=pltpu.CompilerParams(needs_layout_passes=False),
    )
    def emb(x_hbm, table_hbm, pe_hbm, out_hbm, idx_v, rows_v, stg_v, pe_v, gsem, ssem):
        wid = lax.axis_index("s") * NC + lax.axis_index("c")
        base = wid * SEQ_PER_W
        pltpu.sync_copy(pe_hbm, pe_v)

        def start_gather(g, k):
            pltpu.sync_copy(x_hbm.at[base + k], idx_v[g])
            pltpu.async_copy(table_hbm.at[idx_v[g]], rows_v[g], gsem[g])

        start_gather(0, 0)

        def body(i, carry):
            for j in range(2):
                k = i * 2 + j
                g = j
                ga = (j + 1) % 2

                @pl.when(k + 1 < SEQ_PER_W)
                def _():
                    start_gather(ga, k + 1)

                pltpu.make_async_copy(
                    table_hbm.at[idx_v[g]], rows_v[g], gsem[g]
                ).wait()

                @pl.when(k >= 2)
                def _():
                    pltpu.make_async_copy(
                        stg_v[g], out_hbm.at[base + k], ssem[g]
                    ).wait()

                def per_r(r2, c):
                    for ri in range(2):
                        r = r2 * 2 + ri
                        for kk in range(D // 16):
                            sl = pl.ds(16 * kk, 16)
                            stg_v[g][r, sl] = (
                                rows_v[g][r, sl] * SCALE
                                + pe_v[pl.ds(r * D + 16 * kk, 16)]
                            )
                    return c

                lax.fori_loop(0, L // 2, per_r, 0)
                pltpu.async_copy(stg_v[g], out_hbm.at[base + k], ssem[g])
            return carry

        lax.fori_loop(0, SEQ_PER_W // 2, body, 0)

        for j in range(2):
            pltpu.make_async_copy(
                stg_v[j], out_hbm.at[base + SEQ_PER_W - 2 + j], ssem[j]
            ).wait()

    return emb


def kernel(x, table):
    tt = jnp.transpose(table)
    table_p = _build_repack()(tt)
    return _build_lookup()(x, table_p, _PE.reshape(-1))


# speedup vs baseline: 2.0253x; 1.7240x over previous
import functools
import math

import numpy as np
import jax
import jax.numpy as jnp
from jax import lax
from jax.experimental import pallas as pl
from jax.experimental.pallas import tpu as pltpu
from jax.experimental.pallas import tpu_sc as plsc

VOCAB = 1000000
D = 64
B = 4096
L = 200

NC = 2
NS = 16
NW = NC * NS
SEQ_PER_W = B // NW
NBUF = 4

SCALE = 2.0 * math.sqrt(D)


def _make_pe() -> np.ndarray:
    position = np.arange(0, L, dtype=np.float32)[:, None]
    div_even = np.power(10000.0, np.arange(0, D, 2, dtype=np.float32) / D)
    div_odd = np.power(10000.0, np.arange(1, D, 2, dtype=np.float32) / D)
    pe = np.zeros((L, D), dtype=np.float32)
    pe[:, 0::2] = np.sin(position * div_even)
    pe[:, 1::2] = np.cos(position * div_odd)
    return pe


_PE = _make_pe()


@functools.cache
def _build():
    mesh = plsc.VectorSubcoreMesh(
        core_axis_name="c", subcore_axis_name="s", num_cores=NC, num_subcores=NS
    )

    @functools.partial(
        pl.kernel,
        mesh=mesh,
        out_type=jax.ShapeDtypeStruct((B, L, D), jnp.float32),
        scratch_types=[
            [pltpu.VMEM((L,), jnp.int32) for _ in range(NBUF)],
            [pltpu.VMEM((L, D), jnp.float32) for _ in range(NBUF)],
            pltpu.VMEM((L, D), jnp.float32),
            [pltpu.SemaphoreType.DMA for _ in range(NBUF)],
            [pltpu.SemaphoreType.DMA for _ in range(NBUF)],
            [pltpu.SemaphoreType.DMA for _ in range(NBUF)],
        ],
        compiler_params=pltpu.CompilerParams(use_tc_tiling_on_sc=False),
    )
    def emb(x_hbm, table_hbm, pe_hbm, out_hbm, idx_v, rows_v, pe_v, isem, gsem, ssem):
        wid = lax.axis_index("s") * NC + lax.axis_index("c")
        base = wid * SEQ_PER_W
        pltpu.sync_copy(pe_hbm, pe_v)

        def idx_start(b, k):
            pltpu.async_copy(x_hbm.at[base + k], idx_v[b], isem[b])

        def idx_wait(b):
            pltpu.make_async_copy(x_hbm.at[base], idx_v[b], isem[b]).wait()

        def gather_start(b, k):
            pltpu.async_copy(table_hbm.at[idx_v[b]], rows_v[b], gsem[b])

        def gather_wait(b):
            pltpu.make_async_copy(
                table_hbm.at[idx_v[b]], rows_v[b], gsem[b]
            ).wait()

        def store_wait(b):
            pltpu.make_async_copy(rows_v[b], out_hbm.at[base], ssem[b]).wait()

        idx_start(0, 0)
        idx_start(1, 1)
        idx_wait(0)
        gather_start(0, 0)

        def body(i, carry):
            for j in range(NBUF):
                k = i * NBUF + j
                bn = (j + 1) % NBUF
                bi = (j + 2) % NBUF

                @pl.when(k + 2 < SEQ_PER_W)
                def _():
                    idx_start(bi, k + 2)

                @pl.when(k + 1 < SEQ_PER_W)
                def _():
                    idx_wait(bn)

                    @pl.when(k >= NBUF - 1)
                    def _():
                        store_wait(bn)

                    gather_start(bn, k + 1)

                gather_wait(j)

                def per_r(r4, c):
                    for ri in range(4):
                        r = r4 * 4 + ri
                        for kk in range(D // 16):
                            sl = pl.ds(16 * kk, 16)
                            rows_v[j][r, sl] = (
                                rows_v[j][r, sl] * SCALE + pe_v[r, sl]
                            )
                    return c

                lax.fori_loop(0, L // 4, per_r, 0)
                pltpu.async_copy(rows_v[j], out_hbm.at[base + k], ssem[j])
            return carry

        lax.fori_loop(0, SEQ_PER_W // NBUF, body, 0)

        for j in range(NBUF):
            store_wait(j)

    return emb


def kernel(x, table):
    return _build()(x, table, _PE)
